# baseline (speedup 1.0000x reference)
import functools

import jax
import jax.numpy as jnp
from jax import lax
from jax.experimental import pallas as pl
from jax.experimental.pallas import tpu as pltpu
from jax.experimental.pallas import tpu_sc as plsc

N = 2 * 8192 * 2048
NC, NS, L = 2, 16, 16
NW = NC * NS
N_W = N // NW
C = 8 * 1024
G = N_W // C
QG = C // 64


def _body(out_hbm, tgt_hbm, msk_hbm, sums_hbm, cnts_hbm,
          out_v0, out_v1, tgt_v0, tgt_v1, msk_v0, msk_v1,
          st_f, st_c, sems):
    out_v = (out_v0, out_v1)
    tgt_v = (tgt_v0, tgt_v1)
    msk_v = (msk_v0, msk_v1)
    wid = lax.axis_index("s") * NC + lax.axis_index("c")
    base = wid * N_W

    ii = lax.iota(jnp.int32, 16)
    widx = lax.shift_right_logical(ii, 2)
    shifts = lax.shift_left(jnp.bitwise_and(ii, 3), 3)
    perms = [widx + 4 * b for b in range(4)]

    def lane_permute(x, idx):
        return lax.gather(
            x, idx[:, None],
            dimension_numbers=lax.GatherDimensionNumbers(
                offset_dims=(), collapsed_slice_dims=(0,),
                start_index_map=(0,)),
            slice_sizes=(1,),
            mode=lax.GatherScatterMode.PROMISE_IN_BOUNDS)

    def copies(g, slot):
        start = base + g * C
        return (
            pltpu.make_async_copy(out_hbm.at[pl.ds(start, C)], out_v[slot],
                                  sems.at[slot]),
            pltpu.make_async_copy(tgt_hbm.at[pl.ds(start, C)], tgt_v[slot],
                                  sems.at[slot]),
            pltpu.make_async_copy(msk_hbm.at[pl.ds(start, C)],
                                  msk_v[slot], sems.at[slot]),
        )

    def start_chunk(g, slot):
        for c in copies(g, slot):
            c.start()

    def wait_chunk(g, slot):
        for c in copies(g, slot):
            c.wait()

    def compute_chunk(slot, acc, cnt):
        def group_body(q, carry):
            acc2, cp = carry
            w = plsc.bitcast(msk_v[slot][pl.ds(q * 64, 64)], jnp.int32)
            cp = cp + w
            for b in range(4):
                off = q * 64 + b * 16
                d = out_v[slot][pl.ds(off, 16)]
                e = tgt_v[slot][pl.ds(off, 16)]
                wb = lane_permute(w, perms[b])
                m = jnp.bitwise_and(lax.shift_right_logical(wb, shifts), 1)
                mf = m.astype(jnp.float32)
                diff = d - e
                acc2 = acc2 + (diff * mf) * diff
            return acc2, cp

        cp0 = jnp.zeros((16,), jnp.int32)
        acc, cp = lax.fori_loop(0, QG, group_body, (acc, cp0))
        for b in range(4):
            cnt = cnt + jnp.bitwise_and(
                lax.shift_right_logical(cp, 8 * b), 0xFF)
        return acc, cnt

    start_chunk(0, 0)
    start_chunk(1, 1)

    def chunk_pair(gg, carry):
        acc, cnt = carry
        for slot in range(2):
            g = 2 * gg + slot
            wait_chunk(g, slot)
            acc, cnt = compute_chunk(slot, acc, cnt)

            @pl.when(g + 2 < G)
            def _():
                start_chunk(g + 2, slot)
        return acc, cnt

    acc0 = jnp.zeros((16,), jnp.float32)
    cnt0 = jnp.zeros((16,), jnp.int32)
    acc, cnt = lax.fori_loop(0, G // 2, chunk_pair, (acc0, cnt0))

    st_f[...] = acc
    st_c[...] = cnt
    pltpu.sync_copy(st_f, sums_hbm.at[wid])
    pltpu.sync_copy(st_c, cnts_hbm.at[wid])


R_TOT = 2 * 8192
BT = 512


def _tc_body(out_ref, tgt_ref, msk_ref, sums_ref, cnts_ref, acc, cnt):
    @pl.when(pl.program_id(0) == 0)
    def _():
        acc[...] = jnp.zeros_like(acc)
        cnt[...] = jnp.zeros_like(cnt)

    mf = msk_ref[...].astype(jnp.float32)
    diff = out_ref[...] - tgt_ref[...]
    sq = diff * diff * mf
    acc[...] += jnp.sum(sq.reshape(BT // 8, 8, 2048), axis=0)
    cnt[...] += jnp.sum(mf.reshape(BT // 8, 8, 2048), axis=0)

    @pl.when(pl.program_id(0) == pl.num_programs(0) - 1)
    def _():
        sums_ref[...] = acc[...]
        cnts_ref[...] = cnt[...]


def _tc_partial(out2d, tgt2d, msk2d):
    rows = out2d.shape[0]
    grid = rows // BT
    blk = lambda i: (i, 0)
    return pl.pallas_call(
        _tc_body,
        grid=(grid,),
        in_specs=[
            pl.BlockSpec((BT, 2048), blk),
            pl.BlockSpec((BT, 2048), blk),
            pl.BlockSpec((BT, 2048), blk),
        ],
        out_specs=[
            pl.BlockSpec((8, 2048), lambda i: (0, 0)),
            pl.BlockSpec((8, 2048), lambda i: (0, 0)),
        ],
        out_shape=[
            jax.ShapeDtypeStruct((8, 2048), jnp.float32),
            jax.ShapeDtypeStruct((8, 2048), jnp.float32),
        ],
        scratch_shapes=[
            pltpu.VMEM((8, 2048), jnp.float32),
            pltpu.VMEM((8, 2048), jnp.float32),
        ],
        compiler_params=pltpu.CompilerParams(
            dimension_semantics=("arbitrary",)),
    )(out2d, tgt2d, msk2d)


@jax.jit
def kernel(output, target, mask):
    out2d = output.reshape(R_TOT, 2048)
    tgt2d = target.reshape(R_TOT, 2048)
    msk2d = mask.reshape(R_TOT, 2048)
    tsum, tcnt = _tc_partial(out2d, tgt2d, msk2d)
    return jnp.sum(tsum) / jnp.sum(tcnt)


def _sc_kernel_unused(output, target, mask):
    out_flat = output.reshape(-1)
    tgt_flat = target.reshape(-1)
    msk_flat = mask.reshape(-1).view(jnp.uint8)

    mesh = plsc.VectorSubcoreMesh(core_axis_name="c", subcore_axis_name="s")
    sums, cnts = pl.kernel(
        _body,
        mesh=mesh,
        compiler_params=pltpu.CompilerParams(needs_layout_passes=False, use_tc_tiling_on_sc=False),
        out_type=[
            jax.ShapeDtypeStruct((NW, L), jnp.float32),
            jax.ShapeDtypeStruct((NW, L), jnp.int32),
        ],
        scratch_types=[
            pltpu.VMEM((C,), jnp.float32),
            pltpu.VMEM((C,), jnp.float32),
            pltpu.VMEM((C,), jnp.float32),
            pltpu.VMEM((C,), jnp.float32),
            pltpu.VMEM((C,), jnp.uint8),
            pltpu.VMEM((C,), jnp.uint8),
            pltpu.VMEM((L,), jnp.float32),
            pltpu.VMEM((L,), jnp.int32),
            pltpu.SemaphoreType.DMA((2,)),
        ],
    )(out_flat, tgt_flat, msk_flat)

    return jnp.sum(sums) / jnp.sum(cnts).astype(jnp.float32)


# speedup vs baseline: 1.1607x; 1.1607x over previous
import functools

import jax
import jax.numpy as jnp
from jax import lax
from jax.experimental import pallas as pl
from jax.experimental.pallas import tpu as pltpu
from jax.experimental.pallas import tpu_sc as plsc

N = 2 * 8192 * 2048
NC, NS, L = 2, 16, 16
NW = NC * NS
N_W = N // NW
C = 8 * 1024
G = N_W // C
QG = C // 64


def _body(out_hbm, tgt_hbm, msk_hbm, sums_hbm, cnts_hbm,
          out_v0, out_v1, tgt_v0, tgt_v1, msk_v0, msk_v1,
          st_f, st_c, sems):
    out_v = (out_v0, out_v1)
    tgt_v = (tgt_v0, tgt_v1)
    msk_v = (msk_v0, msk_v1)
    wid = lax.axis_index("s") * NC + lax.axis_index("c")
    base = wid * N_W

    ii = lax.iota(jnp.int32, 16)
    widx = lax.shift_right_logical(ii, 2)
    shifts = lax.shift_left(jnp.bitwise_and(ii, 3), 3)
    perms = [widx + 4 * b for b in range(4)]

    def lane_permute(x, idx):
        return lax.gather(
            x, idx[:, None],
            dimension_numbers=lax.GatherDimensionNumbers(
                offset_dims=(), collapsed_slice_dims=(0,),
                start_index_map=(0,)),
            slice_sizes=(1,),
            mode=lax.GatherScatterMode.PROMISE_IN_BOUNDS)

    def copies(g, slot):
        start = base + g * C
        return (
            pltpu.make_async_copy(out_hbm.at[pl.ds(start, C)], out_v[slot],
                                  sems.at[slot]),
            pltpu.make_async_copy(tgt_hbm.at[pl.ds(start, C)], tgt_v[slot],
                                  sems.at[slot]),
            pltpu.make_async_copy(msk_hbm.at[pl.ds(start, C)],
                                  msk_v[slot], sems.at[slot]),
        )

    def start_chunk(g, slot):
        for c in copies(g, slot):
            c.start()

    def wait_chunk(g, slot):
        for c in copies(g, slot):
            c.wait()

    def compute_chunk(slot, acc, cnt):
        def group_body(q, carry):
            acc2, cp = carry
            w = plsc.bitcast(msk_v[slot][pl.ds(q * 64, 64)], jnp.int32)
            cp = cp + w
            for b in range(4):
                off = q * 64 + b * 16
                d = out_v[slot][pl.ds(off, 16)]
                e = tgt_v[slot][pl.ds(off, 16)]
                wb = lane_permute(w, perms[b])
                m = jnp.bitwise_and(lax.shift_right_logical(wb, shifts), 1)
                mf = m.astype(jnp.float32)
                diff = d - e
                acc2 = acc2 + (diff * mf) * diff
            return acc2, cp

        cp0 = jnp.zeros((16,), jnp.int32)
        acc, cp = lax.fori_loop(0, QG, group_body, (acc, cp0))
        for b in range(4):
            cnt = cnt + jnp.bitwise_and(
                lax.shift_right_logical(cp, 8 * b), 0xFF)
        return acc, cnt

    start_chunk(0, 0)
    start_chunk(1, 1)

    def chunk_pair(gg, carry):
        acc, cnt = carry
        for slot in range(2):
            g = 2 * gg + slot
            wait_chunk(g, slot)
            acc, cnt = compute_chunk(slot, acc, cnt)

            @pl.when(g + 2 < G)
            def _():
                start_chunk(g + 2, slot)
        return acc, cnt

    acc0 = jnp.zeros((16,), jnp.float32)
    cnt0 = jnp.zeros((16,), jnp.int32)
    acc, cnt = lax.fori_loop(0, G // 2, chunk_pair, (acc0, cnt0))

    st_f[...] = acc
    st_c[...] = cnt
    pltpu.sync_copy(st_f, sums_hbm.at[wid])
    pltpu.sync_copy(st_c, cnts_hbm.at[wid])


R_TOT = 2 * 8192
BT = 512


def _tc_body(out_ref, tgt_ref, msk_ref, sums_ref, cnts_ref, acc, cnt):
    @pl.when(pl.program_id(0) == 0)
    def _():
        acc[...] = jnp.zeros_like(acc)
        cnt[...] = jnp.zeros_like(cnt)

    ones = jnp.ones((2048,), jnp.float32)
    a = jnp.zeros((8, 2048), jnp.float32)
    c = jnp.zeros((8,), jnp.float32)
    for k in range(BT // 8):
        o = out_ref[pl.ds(8 * k, 8), :]
        t = tgt_ref[pl.ds(8 * k, 8), :]
        m = msk_ref[pl.ds(8 * k, 8), :]
        mf = m.astype(jnp.float32)
        diff = o - t
        a = a + (diff * diff) * mf
        c = c + jnp.dot(mf, ones, preferred_element_type=jnp.float32)
    acc[...] += a
    cnt[...] += c[:, None]

    @pl.when(pl.program_id(0) == pl.num_programs(0) - 1)
    def _():
        sums_ref[...] = acc[...]
        cnts_ref[...] = cnt[...]


def _tc_partial(out2d, tgt2d, msk2d):
    rows = out2d.shape[0]
    grid = rows // BT
    blk = lambda i: (i, 0)
    return pl.pallas_call(
        _tc_body,
        grid=(grid,),
        in_specs=[
            pl.BlockSpec((BT, 2048), blk),
            pl.BlockSpec((BT, 2048), blk),
            pl.BlockSpec((BT, 2048), blk),
        ],
        out_specs=[
            pl.BlockSpec((8, 2048), lambda i: (0, 0)),
            pl.BlockSpec((8, 1), lambda i: (0, 0)),
        ],
        out_shape=[
            jax.ShapeDtypeStruct((8, 2048), jnp.float32),
            jax.ShapeDtypeStruct((8, 1), jnp.float32),
        ],
        scratch_shapes=[
            pltpu.VMEM((8, 2048), jnp.float32),
            pltpu.VMEM((8, 1), jnp.float32),
        ],
        compiler_params=pltpu.CompilerParams(
            dimension_semantics=("arbitrary",)),
    )(out2d, tgt2d, msk2d)


@jax.jit
def kernel(output, target, mask):
    out2d = output.reshape(R_TOT, 2048)
    tgt2d = target.reshape(R_TOT, 2048)
    msk2d = mask.reshape(R_TOT, 2048)
    tsum, tcnt = _tc_partial(out2d, tgt2d, msk2d)
    return jnp.sum(tsum) / jnp.sum(tcnt)


def _sc_kernel_unused(output, target, mask):
    out_flat = output.reshape(-1)
    tgt_flat = target.reshape(-1)
    msk_flat = mask.reshape(-1).view(jnp.uint8)

    mesh = plsc.VectorSubcoreMesh(core_axis_name="c", subcore_axis_name="s")
    sums, cnts = pl.kernel(
        _body,
        mesh=mesh,
        compiler_params=pltpu.CompilerParams(needs_layout_passes=False, use_tc_tiling_on_sc=False),
        out_type=[
            jax.ShapeDtypeStruct((NW, L), jnp.float32),
            jax.ShapeDtypeStruct((NW, L), jnp.int32),
        ],
        scratch_types=[
            pltpu.VMEM((C,), jnp.float32),
            pltpu.VMEM((C,), jnp.float32),
            pltpu.VMEM((C,), jnp.float32),
            pltpu.VMEM((C,), jnp.float32),
            pltpu.VMEM((C,), jnp.uint8),
            pltpu.VMEM((C,), jnp.uint8),
            pltpu.VMEM((L,), jnp.float32),
            pltpu.VMEM((L,), jnp.int32),
            pltpu.SemaphoreType.DMA((2,)),
        ],
    )(out_flat, tgt_flat, msk_flat)

    return jnp.sum(sums) / jnp.sum(cnts).astype(jnp.float32)
